# baseline (speedup 1.0000x reference)
import functools

import jax
import jax.numpy as jnp
from jax import lax
from jax.experimental import pallas as pl
from jax.experimental.pallas import tpu as pltpu
from jax.experimental.pallas import tpu_sc as plsc

E = 1_600_000
EMB = 32
D0, D1, D2, D3 = 6, 7, 3, 23
NROWS = D0 * D1 * D2 * D3

NW = 32
CHUNK = 1280
NIDX = CHUNK // 128
NPAIRS = E // (2 * CHUNK)
BASE_PAIRS = NPAIRS // NW
EXTRA = NPAIRS - BASE_PAIRS * NW

_mesh = plsc.VectorSubcoreMesh(core_axis_name="c", subcore_axis_name="s")


def _perm(v, idx):
    return lax.gather(
        v, idx[:, None],
        lax.GatherDimensionNumbers(
            offset_dims=(), collapsed_slice_dims=(0,), start_index_map=(0,)),
        slice_sizes=(1,),
        mode=lax.GatherScatterMode.PROMISE_IN_BOUNDS)


@functools.partial(
    pl.kernel,
    mesh=_mesh,
    out_type=jax.ShapeDtypeStruct((E, EMB), jnp.float32),
    scratch_types=[
        pltpu.VMEM((2, CHUNK * 4), jnp.int32),
        pltpu.VMEM((2, NIDX, 128), jnp.int32),
        pltpu.VMEM((2, CHUNK, EMB), jnp.float32),
        pltpu.SemaphoreType.DMA,
        pltpu.SemaphoreType.DMA,
        pltpu.SemaphoreType.DMA,
        pltpu.SemaphoreType.DMA,
        pltpu.SemaphoreType.DMA,
        pltpu.SemaphoreType.DMA,
    ],
    compiler_params=pltpu.CompilerParams(use_tc_tiling_on_sc=False),
)
def _lookup(ea_hbm, ct_hbm, out_hbm, raw_v, cidx_v, rows_v,
            sem_i0, sem_i1, sem_g0, sem_g1, sem_o0, sem_o1):
    cid = lax.axis_index("c")
    sid = lax.axis_index("s")
    wid = sid * 2 + cid
    n_pairs = BASE_PAIRS + jnp.where(wid < EXTRA, 1, 0).astype(jnp.int32)

    sem_i = (sem_i0, sem_i1)
    sem_g = (sem_g0, sem_g1)
    sem_o = (sem_o0, sem_o1)

    lanes = lax.iota(jnp.int32, 16)
    m4 = lanes & 3
    coef = jnp.where(m4 == 0, D1 * D2 * D3,
                     jnp.where(m4 == 1, D2 * D3,
                               jnp.where(m4 == 2, D3, 1))).astype(jnp.int32)
    x1 = lanes ^ 1
    x2 = lanes ^ 2
    p4 = m4 * 4
    g0 = (lanes >> 2) == 0
    g1 = (lanes >> 2) == 1
    g2 = (lanes >> 2) == 2

    def pair_body(p, carry):
        pair = wid + NW * p
        base0 = pair * (2 * CHUNK)
        in_dma = []
        for b in range(2):
            base = base0 + b * CHUNK
            in_dma.append(pltpu.async_copy(
                ea_hbm.at[pl.ds(base * 4, CHUNK * 4)],
                raw_v.at[b], sem_i[b]))
        gathers = [[], []]
        for b in range(2):
            base = base0 + b * CHUNK
            in_dma[b].wait()

            def row_body(j, acc, b=b):
                for h in range(8):
                    m = j * 8 + h
                    qs = []
                    for u in range(4):
                        v = raw_v[b, pl.ds((m * 4 + u) * 16, 16)]
                        s = v * coef
                        t = s + _perm(s, x1)
                        q = t + _perm(t, x2)
                        qs.append(_perm(q, p4))
                    cvec = jnp.where(g0, qs[0],
                                     jnp.where(g1, qs[1],
                                               jnp.where(g2, qs[2], qs[3])))
                    cidx_v[b, j, pl.ds(h * 16, 16)] = cvec
                return acc

            lax.fori_loop(0, NIDX, row_body, 0)
            @pl.when(p > 0)
            def _drain(b=b):
                pltpu.make_async_copy(
                    rows_v.at[b], out_hbm.at[pl.ds(0, CHUNK)],
                    sem_o[b]).wait()
            for j in range(NIDX):
                gathers[b].append(pltpu.async_copy(
                    ct_hbm.at[cidx_v.at[b, j]],
                    rows_v.at[b, pl.ds(j * 128, 128)],
                    sem_g[b]))
        for b in range(2):
            base = base0 + b * CHUNK
            for d in gathers[b]:
                d.wait()
            pltpu.async_copy(
                rows_v.at[b], out_hbm.at[pl.ds(base, CHUNK)], sem_o[b])
        return carry

    lax.fori_loop(0, n_pairs, pair_body, 0)
    for b in range(2):
        pltpu.make_async_copy(
            rows_v.at[b], out_hbm.at[pl.ds(0, CHUNK)], sem_o[b]).wait()


def kernel(edge_attr, W0, W1, W2, W3):
    W0z = W0.at[0].set(0.0)
    W1z = W1.at[0].set(0.0)
    W2z = W2.at[0].set(0.0)
    W3z = W3.at[0].set(0.0)
    ct = (W0z[:, None, None, None, :]
          + W1z[None, :, None, None, :]
          + W2z[None, None, :, None, :]
          + W3z[None, None, None, :, :]).reshape(NROWS, EMB)
    ea = edge_attr.reshape(-1)
    return _lookup(ea, ct)


# speedup vs baseline: 2.2428x; 2.2428x over previous
import functools

import jax
import jax.numpy as jnp
from jax import lax
from jax.experimental import pallas as pl
from jax.experimental.pallas import tpu as pltpu
from jax.experimental.pallas import tpu_sc as plsc

E = 1_600_000
EMB = 32
D0, D1, D2, D3 = 6, 7, 3, 23
NROWS = D0 * D1 * D2 * D3

NW = 32
NBLK = E // 128
BPC = 10
CHUNK = BPC * 128
NPAIRS = E // (2 * CHUNK)
BASE_PAIRS = NPAIRS // NW
EXTRA = NPAIRS - BASE_PAIRS * NW

_mesh = plsc.VectorSubcoreMesh(core_axis_name="c", subcore_axis_name="s")


@functools.partial(
    pl.kernel,
    mesh=_mesh,
    out_type=jax.ShapeDtypeStruct((E, EMB), jnp.float32),
    scratch_types=[
        pltpu.VMEM((2, BPC, 4, 128), jnp.int32),
        pltpu.VMEM((2, BPC, 128), jnp.int32),
        pltpu.VMEM((2, CHUNK, EMB), jnp.float32),
        pltpu.SemaphoreType.DMA,
        pltpu.SemaphoreType.DMA,
        pltpu.SemaphoreType.DMA,
        pltpu.SemaphoreType.DMA,
        pltpu.SemaphoreType.DMA,
        pltpu.SemaphoreType.DMA,
    ],
    compiler_params=pltpu.CompilerParams(use_tc_tiling_on_sc=False),
)
def _lookup(ea_hbm, ct_hbm, out_hbm, raw_v, cidx_v, rows_v,
            sem_i0, sem_i1, sem_g0, sem_g1, sem_o0, sem_o1):
    cid = lax.axis_index("c")
    sid = lax.axis_index("s")
    wid = sid * 2 + cid
    n_pairs = BASE_PAIRS + jnp.where(wid < EXTRA, 1, 0).astype(jnp.int32)

    sem_i = (sem_i0, sem_i1)
    sem_g = (sem_g0, sem_g1)
    sem_o = (sem_o0, sem_o1)

    def pair_body(p, carry):
        pair = wid + NW * p
        blk0 = pair * (2 * BPC)
        in_dma = []
        for b in range(2):
            in_dma.append(pltpu.async_copy(
                ea_hbm.at[pl.ds(blk0 + b * BPC, BPC)],
                raw_v.at[b], sem_i[b]))
        gathers = [[], []]
        for b in range(2):
            in_dma[b].wait()
            for j in range(BPC):
                for k in range(8):
                    s = pl.ds(k * 16, 16)
                    c = (raw_v[b, j, 0, s] * (D1 * D2 * D3)
                         + raw_v[b, j, 1, s] * (D2 * D3)
                         + raw_v[b, j, 2, s] * D3
                         + raw_v[b, j, 3, s])
                    cidx_v[b, j, s] = c
            @pl.when(p > 0)
            def _drain(b=b):
                pltpu.make_async_copy(
                    rows_v.at[b], out_hbm.at[pl.ds(0, CHUNK)],
                    sem_o[b]).wait()
            for j in range(BPC):
                gathers[b].append(pltpu.async_copy(
                    ct_hbm.at[cidx_v.at[b, j]],
                    rows_v.at[b, pl.ds(j * 128, 128)],
                    sem_g[b]))
        for b in range(2):
            base = (blk0 + b * BPC) * 128
            for d in gathers[b]:
                d.wait()
            pltpu.async_copy(
                rows_v.at[b], out_hbm.at[pl.ds(base, CHUNK)], sem_o[b])
        return carry

    lax.fori_loop(0, n_pairs, pair_body, 0)
    for b in range(2):
        pltpu.make_async_copy(
            rows_v.at[b], out_hbm.at[pl.ds(0, CHUNK)], sem_o[b]).wait()


def kernel(edge_attr, W0, W1, W2, W3):
    W0z = W0.at[0].set(0.0)
    W1z = W1.at[0].set(0.0)
    W2z = W2.at[0].set(0.0)
    W3z = W3.at[0].set(0.0)
    ct = (W0z[:, None, None, None, :]
          + W1z[None, :, None, None, :]
          + W2z[None, None, :, None, :]
          + W3z[None, None, None, :, :]).reshape(NROWS, EMB)
    ea_b = edge_attr.reshape(NBLK, 128, 4).transpose(0, 2, 1)
    return _lookup(ea_b, ct)


# speedup vs baseline: 4.5589x; 2.0326x over previous
import functools

import jax
import jax.numpy as jnp
from jax import lax
from jax.experimental import pallas as pl
from jax.experimental.pallas import tpu as pltpu
from jax.experimental.pallas import tpu_sc as plsc

E = 1_600_000
EMB = 32
D0, D1, D2, D3 = 6, 7, 3, 23
NROWS = D0 * D1 * D2 * D3

NW = 32
NBLK = E // 128
BPC = 10
CHUNK = BPC * 128
NPAIRS = E // (2 * CHUNK)
BASE_PAIRS = NPAIRS // NW
EXTRA = NPAIRS - BASE_PAIRS * NW

_mesh = plsc.VectorSubcoreMesh(core_axis_name="c", subcore_axis_name="s")


@functools.partial(
    pl.kernel,
    mesh=_mesh,
    out_type=jax.ShapeDtypeStruct((E, EMB), jnp.float32),
    scratch_types=[
        pltpu.VMEM((2, BPC, 4, 128), jnp.int32),
        pltpu.VMEM((2, BPC, 128), jnp.int32),
        pltpu.VMEM((2, CHUNK, EMB), jnp.float32),
        pltpu.VMEM_SHARED((NROWS, EMB), jnp.float32),
        pltpu.SemaphoreType.DMA,
        pltpu.SemaphoreType.DMA,
        pltpu.SemaphoreType.DMA,
        pltpu.SemaphoreType.DMA,
        pltpu.SemaphoreType.DMA,
        pltpu.SemaphoreType.DMA,
    ],
    compiler_params=pltpu.CompilerParams(use_tc_tiling_on_sc=False),
)
def _lookup(ea_hbm, ct_hbm, out_hbm, raw_v, cidx_v, rows_v, ct_sh,
            sem_i0, sem_i1, sem_g0, sem_g1, sem_o0, sem_o1):
    cid = lax.axis_index("c")
    sid = lax.axis_index("s")
    wid = sid * 2 + cid
    n_pairs = BASE_PAIRS + jnp.where(wid < EXTRA, 1, 0).astype(jnp.int32)

    sem_i = (sem_i0, sem_i1)
    sem_g = (sem_g0, sem_g1)
    sem_o = (sem_o0, sem_o1)

    @pl.when(sid == 0)
    def _stage():
        pltpu.sync_copy(ct_hbm, ct_sh)
    plsc.subcore_barrier()

    def pair_body(p, carry):
        pair = wid + NW * p
        blk0 = pair * (2 * BPC)
        in_dma = []
        for b in range(2):
            in_dma.append(pltpu.async_copy(
                ea_hbm.at[pl.ds(blk0 + b * BPC, BPC)],
                raw_v.at[b], sem_i[b]))
        gathers = [[], []]
        for b in range(2):
            in_dma[b].wait()
            for j in range(BPC):
                for k in range(8):
                    s = pl.ds(k * 16, 16)
                    c = (raw_v[b, j, 0, s] * (D1 * D2 * D3)
                         + raw_v[b, j, 1, s] * (D2 * D3)
                         + raw_v[b, j, 2, s] * D3
                         + raw_v[b, j, 3, s])
                    cidx_v[b, j, s] = c
            @pl.when(p > 0)
            def _drain(b=b):
                pltpu.make_async_copy(
                    rows_v.at[b], out_hbm.at[pl.ds(0, CHUNK)],
                    sem_o[b]).wait()
            for j in range(BPC):
                gathers[b].append(pltpu.async_copy(
                    ct_sh.at[cidx_v.at[b, j]],
                    rows_v.at[b, pl.ds(j * 128, 128)],
                    sem_g[b]))
        for b in range(2):
            base = (blk0 + b * BPC) * 128
            for d in gathers[b]:
                d.wait()
            pltpu.async_copy(
                rows_v.at[b], out_hbm.at[pl.ds(base, CHUNK)], sem_o[b])
        return carry

    lax.fori_loop(0, n_pairs, pair_body, 0)
    for b in range(2):
        pltpu.make_async_copy(
            rows_v.at[b], out_hbm.at[pl.ds(0, CHUNK)], sem_o[b]).wait()


def kernel(edge_attr, W0, W1, W2, W3):
    W0z = W0.at[0].set(0.0)
    W1z = W1.at[0].set(0.0)
    W2z = W2.at[0].set(0.0)
    W3z = W3.at[0].set(0.0)
    ct = (W0z[:, None, None, None, :]
          + W1z[None, :, None, None, :]
          + W2z[None, None, :, None, :]
          + W3z[None, None, None, :, :]).reshape(NROWS, EMB)
    ea_b = edge_attr.reshape(NBLK, 128, 4).transpose(0, 2, 1)
    return _lookup(ea_b, ct)
